# Initial kernel scaffold; baseline (speedup 1.0000x reference)
import jax
import jax.numpy as jnp
from jax.experimental import pallas as pl


def kernel(x, edge_index, expander_edge_index, Wq_l, bq_l, Wk_l, bk_l, Wv_l, bv_l, Wo_l, bo_l, Wq_e, bq_e, Wk_e, bk_e, Wv_e, bv_e, Wo_e, bo_e, W1, b1, W2, b2, g1, bt1, g2, bt2, g3, bt3, alpha):
    raise NotImplementedError("write your pallas kernel here")


# speedup vs baseline: 1.0497x; 1.0497x over previous
import functools

import jax
import jax.numpy as jnp
from jax.experimental import pallas as pl
from jax.experimental.pallas import tpu as pltpu

N = 10000
D = 128
H = 8
DH = D // H
BLK = 1000
ACC_W = 144


def _stage1_body(x_ref, g1_ref, bt1_ref,
                 wql_ref, bql_ref, wkl_ref, bkl_ref, wvl_ref, bvl_ref,
                 wqe_ref, bqe_ref, wke_ref, bke_ref, wve_ref, bve_ref,
                 ql_ref, kl_ref, vl_ref, qe_ref, ke_ref, ve_ref):
    x = x_ref[...]
    mu = jnp.mean(x, axis=-1, keepdims=True)
    var = jnp.mean((x - mu) ** 2, axis=-1, keepdims=True)
    xn = (x - mu) * jax.lax.rsqrt(var + 1e-5) * g1_ref[...] + bt1_ref[...]
    f32 = jnp.float32
    ql_ref[...] = jnp.dot(xn, wql_ref[...], preferred_element_type=f32) + bql_ref[...]
    kl_ref[...] = jnp.dot(xn, wkl_ref[...], preferred_element_type=f32) + bkl_ref[...]
    vl_ref[...] = jnp.dot(xn, wvl_ref[...], preferred_element_type=f32) + bvl_ref[...]
    qe_ref[...] = jnp.dot(xn, wqe_ref[...], preferred_element_type=f32) + bqe_ref[...]
    ke_ref[...] = jnp.dot(xn, wke_ref[...], preferred_element_type=f32) + bke_ref[...]
    ve_ref[...] = jnp.dot(xn, wve_ref[...], preferred_element_type=f32) + bve_ref[...]


def _stage1(x, g1, bt1, Wq_l, bq_l, Wk_l, bk_l, Wv_l, bv_l,
            Wq_e, bq_e, Wk_e, bk_e, Wv_e, bv_e):
    row = pl.BlockSpec((BLK, D), lambda i: (i, 0))
    full = pl.BlockSpec((D, D), lambda i: (0, 0))
    vec = pl.BlockSpec((1, D), lambda i: (0, 0))
    out = jax.ShapeDtypeStruct((N, D), jnp.float32)
    return pl.pallas_call(
        _stage1_body,
        grid=(N // BLK,),
        in_specs=[row, vec, vec,
                  full, vec, full, vec, full, vec,
                  full, vec, full, vec, full, vec],
        out_specs=[row] * 6,
        out_shape=[out] * 6,
    )(x, g1.reshape(1, D), bt1.reshape(1, D),
      Wq_l, bq_l.reshape(1, D), Wk_l, bk_l.reshape(1, D), Wv_l, bv_l.reshape(1, D),
      Wq_e, bq_e.reshape(1, D), Wk_e, bk_e.reshape(1, D), Wv_e, bv_e.reshape(1, D))


def _stage3_body(x_ref, accl_ref, acce_ref, a_ref,
                 wol_ref, bol_ref, woe_ref, boe_ref,
                 w1_ref, b1_ref, w2_ref, b2_ref,
                 g2_ref, bt2_ref, g3_ref, bt3_ref, out_ref):
    f32 = jnp.float32
    sel = (jax.lax.broadcasted_iota(jnp.int32, (H, D), 1) // DH
           == jax.lax.broadcasted_iota(jnp.int32, (H, D), 0)).astype(f32)

    def norm_proj(acc_ref, wo_ref, bo_ref):
        acc = acc_ref[0] + acc_ref[1]
        wv = acc[:, :D]
        z = acc[:, D:D + H]
        zw = jnp.dot(z, sel, preferred_element_type=f32)
        o = wv / (zw + 1e-16)
        return jnp.dot(o, wo_ref[...], preferred_element_type=f32) + bo_ref[...]

    x_local = norm_proj(accl_ref, wol_ref, bol_ref)
    x_exp = norm_proj(acce_ref, woe_ref, boe_ref)
    a = a_ref[0, 0]
    x = x_ref[...] + a * x_local + (1.0 - a) * x_exp

    mu = jnp.mean(x, axis=-1, keepdims=True)
    var = jnp.mean((x - mu) ** 2, axis=-1, keepdims=True)
    x = (x - mu) * jax.lax.rsqrt(var + 1e-5) * g2_ref[...] + bt2_ref[...]

    h = jax.nn.gelu(jnp.dot(x, w1_ref[...], preferred_element_type=f32) + b1_ref[...])
    x = x + jnp.dot(h, w2_ref[...], preferred_element_type=f32) + b2_ref[...]

    mu = jnp.mean(x, axis=-1, keepdims=True)
    var = jnp.mean((x - mu) ** 2, axis=-1, keepdims=True)
    out_ref[...] = (x - mu) * jax.lax.rsqrt(var + 1e-5) * g3_ref[...] + bt3_ref[...]


def _stage3(x, acc_l, acc_e, a_sig, Wo_l, bo_l, Wo_e, bo_e,
            W1, b1, W2, b2, g2, bt2, g3, bt3):
    row = pl.BlockSpec((BLK, D), lambda i: (i, 0))
    acc = pl.BlockSpec((2, BLK, ACC_W), lambda i: (0, i, 0))
    vec = pl.BlockSpec((1, D), lambda i: (0, 0))
    return pl.pallas_call(
        _stage3_body,
        grid=(N // BLK,),
        in_specs=[row, acc, acc,
                  pl.BlockSpec((1, 1), lambda i: (0, 0)),
                  pl.BlockSpec((D, D), lambda i: (0, 0)), vec,
                  pl.BlockSpec((D, D), lambda i: (0, 0)), vec,
                  pl.BlockSpec((D, 4 * D), lambda i: (0, 0)),
                  pl.BlockSpec((1, 4 * D), lambda i: (0, 0)),
                  pl.BlockSpec((4 * D, D), lambda i: (0, 0)), vec,
                  vec, vec, vec, vec],
        out_specs=row,
        out_shape=jax.ShapeDtypeStruct((N, D), jnp.float32),
    )(x, acc_l, acc_e, a_sig.reshape(1, 1),
      Wo_l, bo_l.reshape(1, D), Wo_e, bo_e.reshape(1, D),
      W1, b1.reshape(1, 4 * D), W2, b2.reshape(1, D),
      g2.reshape(1, D), bt2.reshape(1, D), g3.reshape(1, D), bt3.reshape(1, D))


def _edge_acc_xla(q, k, v, ei):
    src, dst = ei[0], ei[1]
    qh = q.reshape(N, H, DH)
    kh = k.reshape(N, H, DH)
    vh = v.reshape(N, H, DH)
    s = jnp.sum(qh[dst] * kh[src], axis=-1) / (DH ** 0.5)
    w = jnp.exp(s)
    ow = jax.ops.segment_sum(w[:, :, None] * vh[src], dst, num_segments=N)
    z = jax.ops.segment_sum(w, dst, num_segments=N)
    acc = jnp.concatenate([ow.reshape(N, D), z, jnp.zeros((N, H), jnp.float32)],
                          axis=1)
    return jnp.stack([acc, jnp.zeros_like(acc)])


def kernel(x, edge_index, expander_edge_index,
           Wq_l, bq_l, Wk_l, bk_l, Wv_l, bv_l, Wo_l, bo_l,
           Wq_e, bq_e, Wk_e, bk_e, Wv_e, bv_e, Wo_e, bo_e,
           W1, b1, W2, b2, g1, bt1, g2, bt2, g3, bt3, alpha):
    ql, kl, vl, qe, ke, ve = _stage1(
        x, g1, bt1, Wq_l, bq_l, Wk_l, bk_l, Wv_l, bv_l,
        Wq_e, bq_e, Wk_e, bk_e, Wv_e, bv_e)
    acc_l = _edge_acc_xla(ql, kl, vl, edge_index)
    acc_e = _edge_acc_xla(qe, ke, ve, expander_edge_index)
    a_sig = jax.nn.sigmoid(alpha)
    return _stage3(x, acc_l, acc_e, a_sig, Wo_l, bo_l, Wo_e, bo_e,
                   W1, b1, W2, b2, g2, bt2, g3, bt3)


# speedup vs baseline: 12.7212x; 12.1193x over previous
import dataclasses
import functools

import jax
import jax.numpy as jnp
from jax import lax
from jax.experimental import pallas as pl
from jax.experimental.pallas import tpu as pltpu
from jax.experimental.pallas import tpu_sc as plsc

N = 10000
D = 128
H = 8
DH = D // H
BLK = 1000
ACC_W = 144

SC_CORES = 2
SC_SUBCORES = 16
EC = 80
EG = EC // 16
ZPK = N // 16
TBL = 10640


def _stage1_body(x_ref, g1_ref, bt1_ref,
                 wql_ref, bql_ref, wkl_ref, bkl_ref, wvl_ref, bvl_ref,
                 wqe_ref, bqe_ref, wke_ref, bke_ref, wve_ref, bve_ref,
                 ql_ref, kl_ref, vl_ref, qe_ref, ke_ref, ve_ref):
    x = x_ref[...]
    mu = jnp.mean(x, axis=-1, keepdims=True)
    var = jnp.mean((x - mu) ** 2, axis=-1, keepdims=True)
    xn = (x - mu) * jax.lax.rsqrt(var + 1e-5) * g1_ref[...] + bt1_ref[...]
    f32 = jnp.float32
    ql_ref[...] = jnp.dot(xn, wql_ref[...], preferred_element_type=f32) + bql_ref[...]
    kl_ref[...] = jnp.dot(xn, wkl_ref[...], preferred_element_type=f32) + bkl_ref[...]
    vl_ref[...] = jnp.dot(xn, wvl_ref[...], preferred_element_type=f32) + bvl_ref[...]
    qe_ref[...] = jnp.dot(xn, wqe_ref[...], preferred_element_type=f32) + bqe_ref[...]
    ke_ref[...] = jnp.dot(xn, wke_ref[...], preferred_element_type=f32) + bke_ref[...]
    ve_ref[...] = jnp.dot(xn, wve_ref[...], preferred_element_type=f32) + bve_ref[...]


def _stage1(x, g1, bt1, Wq_l, bq_l, Wk_l, bk_l, Wv_l, bv_l,
            Wq_e, bq_e, Wk_e, bk_e, Wv_e, bv_e):
    row = pl.BlockSpec((BLK, D), lambda i: (i, 0))
    full = pl.BlockSpec((D, D), lambda i: (0, 0))
    vec = pl.BlockSpec((1, D), lambda i: (0, 0))
    out = jax.ShapeDtypeStruct((N, D), jnp.float32)
    return pl.pallas_call(
        _stage1_body,
        grid=(N // BLK,),
        in_specs=[row, vec, vec,
                  full, vec, full, vec, full, vec,
                  full, vec, full, vec, full, vec],
        out_specs=[row] * 6,
        out_shape=[out] * 6,
    )(x, g1.reshape(1, D), bt1.reshape(1, D),
      Wq_l, bq_l.reshape(1, D), Wk_l, bk_l.reshape(1, D), Wv_l, bv_l.reshape(1, D),
      Wq_e, bq_e.reshape(1, D), Wk_e, bk_e.reshape(1, D), Wv_e, bv_e.reshape(1, D))


def _stage3_body(x_ref, wvl_ref, zl_ref, wve_ref, ze_ref, a_ref,
                 wol_ref, bol_ref, woe_ref, boe_ref,
                 w1_ref, b1_ref, w2_ref, b2_ref,
                 g2_ref, bt2_ref, g3_ref, bt3_ref, out_ref):
    f32 = jnp.float32
    sel = (jax.lax.broadcasted_iota(jnp.int32, (H, D), 1) // DH
           == jax.lax.broadcasted_iota(jnp.int32, (H, D), 0)).astype(f32)

    def norm_proj(wv_ref, z_ref, wo_ref, bo_ref):
        wv = wv_ref[0] + wv_ref[1]
        z = jnp.sum(z_ref[...], axis=0)
        zw = jnp.dot(z, sel, preferred_element_type=f32)
        o = wv / (zw + 1e-16)
        return jnp.dot(o, wo_ref[...], preferred_element_type=f32) + bo_ref[...]

    x_local = norm_proj(wvl_ref, zl_ref, wol_ref, bol_ref)
    x_exp = norm_proj(wve_ref, ze_ref, woe_ref, boe_ref)
    a = a_ref[0, 0]
    x = x_ref[...] + a * x_local + (1.0 - a) * x_exp

    mu = jnp.mean(x, axis=-1, keepdims=True)
    var = jnp.mean((x - mu) ** 2, axis=-1, keepdims=True)
    x = (x - mu) * jax.lax.rsqrt(var + 1e-5) * g2_ref[...] + bt2_ref[...]

    h = jax.nn.gelu(jnp.dot(x, w1_ref[...], preferred_element_type=f32) + b1_ref[...])
    x = x + jnp.dot(h, w2_ref[...], preferred_element_type=f32) + b2_ref[...]

    mu = jnp.mean(x, axis=-1, keepdims=True)
    var = jnp.mean((x - mu) ** 2, axis=-1, keepdims=True)
    out_ref[...] = (x - mu) * jax.lax.rsqrt(var + 1e-5) * g3_ref[...] + bt3_ref[...]


def _stage3(x, wv_l, z_l, wv_e, z_e, a_sig, Wo_l, bo_l, Wo_e, bo_e,
            W1, b1, W2, b2, g2, bt2, g3, bt3):
    row = pl.BlockSpec((BLK, D), lambda i: (i, 0))
    wvs = pl.BlockSpec((2, BLK, D), lambda i: (0, i, 0))
    zs = pl.BlockSpec((SC_CORES, BLK, H), lambda i: (0, i, 0))
    vec = pl.BlockSpec((1, D), lambda i: (0, 0))
    return pl.pallas_call(
        _stage3_body,
        grid=(N // BLK,),
        in_specs=[row, wvs, zs, wvs, zs,
                  pl.BlockSpec((1, 1), lambda i: (0, 0)),
                  pl.BlockSpec((D, D), lambda i: (0, 0)), vec,
                  pl.BlockSpec((D, D), lambda i: (0, 0)), vec,
                  pl.BlockSpec((D, 4 * D), lambda i: (0, 0)),
                  pl.BlockSpec((1, 4 * D), lambda i: (0, 0)),
                  pl.BlockSpec((4 * D, D), lambda i: (0, 0)), vec,
                  vec, vec, vec, vec],
        out_specs=row,
        out_shape=jax.ShapeDtypeStruct((N, D), jnp.float32),
    )(x, wv_l, z_l, wv_e, z_e, a_sig.reshape(1, 1),
      Wo_l, bo_l.reshape(1, D), Wo_e, bo_e.reshape(1, D),
      W1, b1.reshape(1, 4 * D), W2, b2.reshape(1, D),
      g2.reshape(1, D), bt2.reshape(1, D), g3.reshape(1, D), bt3.reshape(1, D))


def _edge_phase(q_hbm, k_hbm, v_hbm, dst_hbm, src_hbm, out_tbl,
                dsti, srci, zidxi, qrows, krows, vrows, zrows, table,
                sem_q, sem_k, sem_v, chunks_per_core, ci, sid):
    iters = (chunks_per_core + SC_SUBCORES - 1) // SC_SUBCORES
    zero16 = jnp.zeros((16,), jnp.float32)
    iota = lax.iota(jnp.int32, 16)

    @pl.loop(0, EC)
    def _(r):
        for j in range(D // 16):
            zrows[r, pl.ds(j * 16, 16)] = zero16

    @pl.loop(0, (TBL // EC + SC_SUBCORES - 1) // SC_SUBCORES)
    def _(i):
        strip = sid + i * SC_SUBCORES

        @pl.when(strip < TBL // EC)
        def _():
            pltpu.sync_copy(zrows, table.at[pl.ds(strip * EC, EC)])

    plsc.subcore_barrier()

    @pl.loop(0, iters)
    def _(it):
        c = sid + it * SC_SUBCORES

        @pl.when(c < chunks_per_core)
        def _():
            ebase = (ci * chunks_per_core + c) * EC
            pltpu.sync_copy(dst_hbm.at[pl.ds(ebase, EC)], dsti)
            pltpu.sync_copy(src_hbm.at[pl.ds(ebase, EC)], srci)
            cq = pltpu.async_copy(q_hbm.at[dsti], qrows, sem_q)
            ck = pltpu.async_copy(k_hbm.at[srci], krows, sem_k)
            cv = pltpu.async_copy(v_hbm.at[srci], vrows, sem_v)
            cq.wait()
            ck.wait()
            cv.wait()

            @pl.loop(0, EG)
            def _(g):
                rows = g * 16 + iota
                dvec = dsti[pl.ds(g * 16, 16)]
                zidxi[pl.ds(g * 16, 16)] = (
                    N + lax.shift_right_logical(dvec, 4))
                zcol0 = (dvec & 15) * 8
                for h in range(H):
                    acc = zero16
                    for t in range(DH):
                        col = jnp.full((16,), h * DH + t, jnp.int32)
                        qc = plsc.load_gather(qrows, [rows, col])
                        kc = plsc.load_gather(krows, [rows, col])
                        acc = acc + qc * kc
                    w = jnp.exp(acc * (1.0 / (DH ** 0.5)))
                    plsc.store_scatter(zrows, [rows, zcol0 + h], w)
                    for t in range(DH):
                        col = jnp.full((16,), h * DH + t, jnp.int32)
                        vc = plsc.load_gather(vrows, [rows, col])
                        plsc.store_scatter(vrows, [rows, col], w * vc)

            pltpu.sync_copy(vrows, table.at[dsti], add=True)
            pltpu.sync_copy(zrows, table.at[zidxi], add=True)

            @pl.loop(0, EG)
            def _(g):
                rows = g * 16 + iota
                dvec = dsti[pl.ds(g * 16, 16)]
                zcol0 = (dvec & 15) * 8
                for h in range(H):
                    plsc.store_scatter(zrows, [rows, zcol0 + h], zero16)

    plsc.subcore_barrier()

    @pl.loop(0, (TBL // EC + SC_SUBCORES - 1) // SC_SUBCORES)
    def _(i):
        strip = sid + i * SC_SUBCORES

        @pl.when(strip < TBL // EC)
        def _():
            pltpu.sync_copy(table.at[pl.ds(strip * EC, EC)],
                            out_tbl.at[ci, pl.ds(strip * EC, EC)])


def _edges_body(ql, kl, vl, qe, ke, ve, dl, sl, de, se, otl, ote,
                dsti, srci, zidxi, qrows, krows, vrows, zrows, table,
                sem_q, sem_k, sem_v, chunks_l, chunks_e):
    ci = lax.axis_index("c")
    sid = lax.axis_index("s")
    scratch = (dsti, srci, zidxi, qrows, krows, vrows, zrows, table,
               sem_q, sem_k, sem_v)
    _edge_phase(ql, kl, vl, dl, sl, otl, *scratch, chunks_l, ci, sid)
    plsc.subcore_barrier()
    _edge_phase(qe, ke, ve, de, se, ote, *scratch, chunks_e, ci, sid)


def _edge_acc_sc(ql, kl, vl, qe, ke, ve, ei_l, ei_e, ne_l, ne_e):
    mesh = plsc.VectorSubcoreMesh(core_axis_name="c", subcore_axis_name="s")
    cp = pltpu.CompilerParams()
    if "needs_layout_passes" in pltpu.CompilerParams.__dataclass_fields__:
        cp = dataclasses.replace(cp, needs_layout_passes=False)

    body = functools.partial(_edges_body,
                             chunks_l=ne_l // SC_CORES // EC,
                             chunks_e=ne_e // SC_CORES // EC)
    tbl_t = jax.ShapeDtypeStruct((SC_CORES, TBL, D), jnp.float32)
    kern = pl.kernel(
        body,
        compiler_params=cp,
        out_type=[tbl_t, tbl_t],
        mesh=mesh,
        scratch_types=[
            pltpu.VMEM((EC,), jnp.int32),
            pltpu.VMEM((EC,), jnp.int32),
            pltpu.VMEM((EC,), jnp.int32),
            pltpu.VMEM((EC, D), jnp.float32),
            pltpu.VMEM((EC, D), jnp.float32),
            pltpu.VMEM((EC, D), jnp.float32),
            pltpu.VMEM((EC, D), jnp.float32),
            pltpu.VMEM_SHARED((TBL, D), jnp.float32),
            pltpu.SemaphoreType.DMA,
            pltpu.SemaphoreType.DMA,
            pltpu.SemaphoreType.DMA,
        ],
    )
    return kern(ql, kl, vl, qe, ke, ve, ei_l[1], ei_l[0], ei_e[1], ei_e[0])


def kernel(x, edge_index, expander_edge_index,
           Wq_l, bq_l, Wk_l, bk_l, Wv_l, bv_l, Wo_l, bo_l,
           Wq_e, bq_e, Wk_e, bk_e, Wv_e, bv_e, Wo_e, bo_e,
           W1, b1, W2, b2, g1, bt1, g2, bt2, g3, bt3, alpha):
    ql, kl, vl, qe, ke, ve = _stage1(
        x, g1, bt1, Wq_l, bq_l, Wk_l, bk_l, Wv_l, bv_l,
        Wq_e, bq_e, Wk_e, bk_e, Wv_e, bv_e)
    otl, ote = _edge_acc_sc(
        ql, kl, vl, qe, ke, ve, edge_index, expander_edge_index,
        320000, 40000)
    wv_l = otl[:, :N, :]
    wv_e = ote[:, :N, :]
    z_l = otl[:, N:N + ZPK, :].reshape(SC_CORES, N, H)
    z_e = ote[:, N:N + ZPK, :].reshape(SC_CORES, N, H)
    a_sig = jax.nn.sigmoid(alpha)
    return _stage3(x, wv_l, z_l, wv_e, z_e, a_sig, Wo_l, bo_l, Wo_e, bo_e,
                   W1, b1, W2, b2, g2, bt2, g3, bt3)


# speedup vs baseline: 16.0981x; 1.2654x over previous
import dataclasses
import functools

import jax
import jax.numpy as jnp
from jax import lax
from jax.experimental import pallas as pl
from jax.experimental.pallas import tpu as pltpu
from jax.experimental.pallas import tpu_sc as plsc

N = 10000
D = 128
H = 8
DH = D // H
BLK = 1000
ACC_W = 144

SC_CORES = 2
SC_SUBCORES = 16
EC = 80
EG = EC // 16
ZPK = N // 16
TBL = 10640


def _stage1_body(x_ref, g1_ref, bt1_ref,
                 wql_ref, bql_ref, wkl_ref, bkl_ref, wvl_ref, bvl_ref,
                 wqe_ref, bqe_ref, wke_ref, bke_ref, wve_ref, bve_ref,
                 ql_ref, kl_ref, vl_ref, qe_ref, ke_ref, ve_ref):
    x = x_ref[...]
    mu = jnp.mean(x, axis=-1, keepdims=True)
    var = jnp.mean((x - mu) ** 2, axis=-1, keepdims=True)
    xn = (x - mu) * jax.lax.rsqrt(var + 1e-5) * g1_ref[...] + bt1_ref[...]
    f32 = jnp.float32
    ql_ref[...] = jnp.dot(xn, wql_ref[...], preferred_element_type=f32) + bql_ref[...]
    kl_ref[...] = jnp.dot(xn, wkl_ref[...], preferred_element_type=f32) + bkl_ref[...]
    vl_ref[...] = jnp.dot(xn, wvl_ref[...], preferred_element_type=f32) + bvl_ref[...]
    qe_ref[...] = jnp.dot(xn, wqe_ref[...], preferred_element_type=f32) + bqe_ref[...]
    ke_ref[...] = jnp.dot(xn, wke_ref[...], preferred_element_type=f32) + bke_ref[...]
    ve_ref[...] = jnp.dot(xn, wve_ref[...], preferred_element_type=f32) + bve_ref[...]


def _stage1(x, g1, bt1, Wq_l, bq_l, Wk_l, bk_l, Wv_l, bv_l,
            Wq_e, bq_e, Wk_e, bk_e, Wv_e, bv_e):
    row = pl.BlockSpec((BLK, D), lambda i: (i, 0))
    full = pl.BlockSpec((D, D), lambda i: (0, 0))
    vec = pl.BlockSpec((1, D), lambda i: (0, 0))
    out = jax.ShapeDtypeStruct((N, D), jnp.float32)
    return pl.pallas_call(
        _stage1_body,
        grid=(N // BLK,),
        in_specs=[row, vec, vec,
                  full, vec, full, vec, full, vec,
                  full, vec, full, vec, full, vec],
        out_specs=[row] * 6,
        out_shape=[out] * 6,
    )(x, g1.reshape(1, D), bt1.reshape(1, D),
      Wq_l, bq_l.reshape(1, D), Wk_l, bk_l.reshape(1, D), Wv_l, bv_l.reshape(1, D),
      Wq_e, bq_e.reshape(1, D), Wk_e, bk_e.reshape(1, D), Wv_e, bv_e.reshape(1, D))


def _stage3_body(x_ref, wvl_ref, zl_ref, wve_ref, ze_ref, a_ref,
                 wol_ref, bol_ref, woe_ref, boe_ref,
                 w1_ref, b1_ref, w2_ref, b2_ref,
                 g2_ref, bt2_ref, g3_ref, bt3_ref, out_ref):
    f32 = jnp.float32
    sel = (jax.lax.broadcasted_iota(jnp.int32, (H, D), 1) // DH
           == jax.lax.broadcasted_iota(jnp.int32, (H, D), 0)).astype(f32)

    def norm_proj(wv_ref, z_ref, wo_ref, bo_ref):
        wv = wv_ref[0] + wv_ref[1]
        z = jnp.sum(z_ref[...], axis=0)
        zw = jnp.dot(z, sel, preferred_element_type=f32)
        o = wv / (zw + 1e-16)
        return jnp.dot(o, wo_ref[...], preferred_element_type=f32) + bo_ref[...]

    x_local = norm_proj(wvl_ref, zl_ref, wol_ref, bol_ref)
    x_exp = norm_proj(wve_ref, ze_ref, woe_ref, boe_ref)
    a = a_ref[0, 0]
    x = x_ref[...] + a * x_local + (1.0 - a) * x_exp

    mu = jnp.mean(x, axis=-1, keepdims=True)
    var = jnp.mean((x - mu) ** 2, axis=-1, keepdims=True)
    x = (x - mu) * jax.lax.rsqrt(var + 1e-5) * g2_ref[...] + bt2_ref[...]

    h = jax.nn.gelu(jnp.dot(x, w1_ref[...], preferred_element_type=f32) + b1_ref[...])
    x = x + jnp.dot(h, w2_ref[...], preferred_element_type=f32) + b2_ref[...]

    mu = jnp.mean(x, axis=-1, keepdims=True)
    var = jnp.mean((x - mu) ** 2, axis=-1, keepdims=True)
    out_ref[...] = (x - mu) * jax.lax.rsqrt(var + 1e-5) * g3_ref[...] + bt3_ref[...]


def _stage3(x, wv_l, z_l, wv_e, z_e, a_sig, Wo_l, bo_l, Wo_e, bo_e,
            W1, b1, W2, b2, g2, bt2, g3, bt3):
    row = pl.BlockSpec((BLK, D), lambda i: (i, 0))
    wvs = pl.BlockSpec((2, BLK, D), lambda i: (0, i, 0))
    zs = pl.BlockSpec((SC_CORES, BLK, H), lambda i: (0, i, 0))
    vec = pl.BlockSpec((1, D), lambda i: (0, 0))
    return pl.pallas_call(
        _stage3_body,
        grid=(N // BLK,),
        in_specs=[row, wvs, zs, wvs, zs,
                  pl.BlockSpec((1, 1), lambda i: (0, 0)),
                  pl.BlockSpec((D, D), lambda i: (0, 0)), vec,
                  pl.BlockSpec((D, D), lambda i: (0, 0)), vec,
                  pl.BlockSpec((D, 4 * D), lambda i: (0, 0)),
                  pl.BlockSpec((1, 4 * D), lambda i: (0, 0)),
                  pl.BlockSpec((4 * D, D), lambda i: (0, 0)), vec,
                  vec, vec, vec, vec],
        out_specs=row,
        out_shape=jax.ShapeDtypeStruct((N, D), jnp.float32),
    )(x, wv_l, z_l, wv_e, z_e, a_sig.reshape(1, 1),
      Wo_l, bo_l.reshape(1, D), Wo_e, bo_e.reshape(1, D),
      W1, b1.reshape(1, 4 * D), W2, b2.reshape(1, D),
      g2.reshape(1, D), bt2.reshape(1, D), g3.reshape(1, D), bt3.reshape(1, D))


def _edge_phase(q_hbm, k_hbm, v_hbm, dst_hbm, src_hbm, out_tbl,
                dsti, srci, zidxi, qrows, krows, vrows, zrows, table,
                sem_q, sem_k, sem_v, chunks_per_core, ci, sid):
    iters = (chunks_per_core + SC_SUBCORES - 1) // SC_SUBCORES
    zero16 = jnp.zeros((16,), jnp.float32)
    iota = lax.iota(jnp.int32, 16)

    @pl.loop(0, EC)
    def _(r):
        for j in range(D // 16):
            zrows[r, pl.ds(j * 16, 16)] = zero16

    @pl.loop(0, (TBL // EC + SC_SUBCORES - 1) // SC_SUBCORES)
    def _(i):
        strip = sid + i * SC_SUBCORES

        @pl.when(strip < TBL // EC)
        def _():
            pltpu.sync_copy(zrows, table.at[pl.ds(strip * EC, EC)])

    plsc.subcore_barrier()

    @pl.loop(0, iters)
    def _(it):
        c = sid + it * SC_SUBCORES

        @pl.when(c < chunks_per_core)
        def _():
            ebase = (ci * chunks_per_core + c) * EC
            pltpu.sync_copy(dst_hbm.at[pl.ds(ebase, EC)], dsti)
            pltpu.sync_copy(src_hbm.at[pl.ds(ebase, EC)], srci)
            cq = pltpu.async_copy(q_hbm.at[dsti], qrows, sem_q)
            ck = pltpu.async_copy(k_hbm.at[srci], krows, sem_k)
            cv = pltpu.async_copy(v_hbm.at[srci], vrows, sem_v)
            cq.wait()
            ck.wait()
            cv.wait()

            @pl.loop(0, EG)
            def _(g):
                rows = g * 16 + iota
                dvec = dsti[pl.ds(g * 16, 16)]
                zidxi[pl.ds(g * 16, 16)] = (
                    N + lax.shift_right_logical(dvec, 4))
                zcol0 = (dvec & 15) * 8
                cols = [jnp.full((16,), d, jnp.int32) for d in range(D)]
                for h in range(H):
                    b = h * DH
                    qs = [plsc.load_gather(qrows, [rows, cols[b + t]])
                          for t in range(DH)]
                    ks = [plsc.load_gather(krows, [rows, cols[b + t]])
                          for t in range(DH)]
                    ps = [q * k for q, k in zip(qs, ks)]
                    while len(ps) > 1:
                        ps = [a + c for a, c in zip(ps[::2], ps[1::2])]
                    w = jnp.exp(ps[0] * (1.0 / (DH ** 0.5)))
                    plsc.store_scatter(zrows, [rows, zcol0 + h], w)
                    vs = [plsc.load_gather(vrows, [rows, cols[b + t]])
                          for t in range(DH)]
                    for t in range(DH):
                        plsc.store_scatter(vrows, [rows, cols[b + t]],
                                           w * vs[t])

            pltpu.sync_copy(vrows, table.at[dsti], add=True)
            pltpu.sync_copy(zrows, table.at[zidxi], add=True)

            @pl.loop(0, EG)
            def _(g):
                rows = g * 16 + iota
                dvec = dsti[pl.ds(g * 16, 16)]
                zcol0 = (dvec & 15) * 8
                for h in range(H):
                    plsc.store_scatter(zrows, [rows, zcol0 + h], zero16)

    plsc.subcore_barrier()

    @pl.loop(0, (TBL // EC + SC_SUBCORES - 1) // SC_SUBCORES)
    def _(i):
        strip = sid + i * SC_SUBCORES

        @pl.when(strip < TBL // EC)
        def _():
            pltpu.sync_copy(table.at[pl.ds(strip * EC, EC)],
                            out_tbl.at[ci, pl.ds(strip * EC, EC)])


def _edges_body(ql, kl, vl, qe, ke, ve, dl, sl, de, se, otl, ote,
                dsti, srci, zidxi, qrows, krows, vrows, zrows, table,
                sem_q, sem_k, sem_v, chunks_l, chunks_e):
    ci = lax.axis_index("c")
    sid = lax.axis_index("s")
    scratch = (dsti, srci, zidxi, qrows, krows, vrows, zrows, table,
               sem_q, sem_k, sem_v)
    _edge_phase(ql, kl, vl, dl, sl, otl, *scratch, chunks_l, ci, sid)
    plsc.subcore_barrier()
    _edge_phase(qe, ke, ve, de, se, ote, *scratch, chunks_e, ci, sid)


def _edge_acc_sc(ql, kl, vl, qe, ke, ve, ei_l, ei_e, ne_l, ne_e):
    mesh = plsc.VectorSubcoreMesh(core_axis_name="c", subcore_axis_name="s")
    cp = pltpu.CompilerParams()
    if "needs_layout_passes" in pltpu.CompilerParams.__dataclass_fields__:
        cp = dataclasses.replace(cp, needs_layout_passes=False)

    body = functools.partial(_edges_body,
                             chunks_l=ne_l // SC_CORES // EC,
                             chunks_e=ne_e // SC_CORES // EC)
    tbl_t = jax.ShapeDtypeStruct((SC_CORES, TBL, D), jnp.float32)
    kern = pl.kernel(
        body,
        compiler_params=cp,
        out_type=[tbl_t, tbl_t],
        mesh=mesh,
        scratch_types=[
            pltpu.VMEM((EC,), jnp.int32),
            pltpu.VMEM((EC,), jnp.int32),
            pltpu.VMEM((EC,), jnp.int32),
            pltpu.VMEM((EC, D), jnp.float32),
            pltpu.VMEM((EC, D), jnp.float32),
            pltpu.VMEM((EC, D), jnp.float32),
            pltpu.VMEM((EC, D), jnp.float32),
            pltpu.VMEM_SHARED((TBL, D), jnp.float32),
            pltpu.SemaphoreType.DMA,
            pltpu.SemaphoreType.DMA,
            pltpu.SemaphoreType.DMA,
        ],
    )
    return kern(ql, kl, vl, qe, ke, ve, ei_l[1], ei_l[0], ei_e[1], ei_e[0])


def kernel(x, edge_index, expander_edge_index,
           Wq_l, bq_l, Wk_l, bk_l, Wv_l, bv_l, Wo_l, bo_l,
           Wq_e, bq_e, Wk_e, bk_e, Wv_e, bv_e, Wo_e, bo_e,
           W1, b1, W2, b2, g1, bt1, g2, bt2, g3, bt3, alpha):
    ql, kl, vl, qe, ke, ve = _stage1(
        x, g1, bt1, Wq_l, bq_l, Wk_l, bk_l, Wv_l, bv_l,
        Wq_e, bq_e, Wk_e, bk_e, Wv_e, bv_e)
    otl, ote = _edge_acc_sc(
        ql, kl, vl, qe, ke, ve, edge_index, expander_edge_index,
        320000, 40000)
    wv_l = otl[:, :N, :]
    wv_e = ote[:, :N, :]
    z_l = otl[:, N:N + ZPK, :].reshape(SC_CORES, N, H)
    z_e = ote[:, N:N + ZPK, :].reshape(SC_CORES, N, H)
    a_sig = jax.nn.sigmoid(alpha)
    return _stage3(x, wv_l, z_l, wv_e, z_e, a_sig, Wo_l, bo_l, Wo_e, bo_e,
                   W1, b1, W2, b2, g2, bt2, g3, bt3)


# speedup vs baseline: 50.7072x; 3.1499x over previous
import dataclasses
import functools

import jax
import jax.numpy as jnp
from jax import lax
from jax.experimental import pallas as pl
from jax.experimental.pallas import tpu as pltpu
from jax.experimental.pallas import tpu_sc as plsc

N = 10000
D = 128
H = 8
DH = D // H
BLK = 1000
ACC_W = 144

SC_CORES = 2
SC_SUBCORES = 16
EC = 80
EG = EC // 16
ZPK = N // 16
TBL = 10640


def _stage1_body(x_ref, g1_ref, bt1_ref,
                 wql_ref, bql_ref, wkl_ref, bkl_ref, wvl_ref, bvl_ref,
                 wqe_ref, bqe_ref, wke_ref, bke_ref, wve_ref, bve_ref,
                 ql_ref, kl_ref, vl_ref, qe_ref, ke_ref, ve_ref):
    x = x_ref[...]
    mu = jnp.mean(x, axis=-1, keepdims=True)
    var = jnp.mean((x - mu) ** 2, axis=-1, keepdims=True)
    xn = (x - mu) * jax.lax.rsqrt(var + 1e-5) * g1_ref[...] + bt1_ref[...]
    f32 = jnp.float32
    ql_ref[...] = jnp.dot(xn, wql_ref[...], preferred_element_type=f32) + bql_ref[...]
    kl_ref[...] = jnp.dot(xn, wkl_ref[...], preferred_element_type=f32) + bkl_ref[...]
    vl_ref[...] = jnp.dot(xn, wvl_ref[...], preferred_element_type=f32) + bvl_ref[...]
    qe_ref[...] = jnp.dot(xn, wqe_ref[...], preferred_element_type=f32) + bqe_ref[...]
    ke_ref[...] = jnp.dot(xn, wke_ref[...], preferred_element_type=f32) + bke_ref[...]
    ve_ref[...] = jnp.dot(xn, wve_ref[...], preferred_element_type=f32) + bve_ref[...]


def _stage1(x, g1, bt1, Wq_l, bq_l, Wk_l, bk_l, Wv_l, bv_l,
            Wq_e, bq_e, Wk_e, bk_e, Wv_e, bv_e):
    row = pl.BlockSpec((BLK, D), lambda i: (i, 0))
    full = pl.BlockSpec((D, D), lambda i: (0, 0))
    vec = pl.BlockSpec((1, D), lambda i: (0, 0))
    out = jax.ShapeDtypeStruct((N, D), jnp.float32)
    return pl.pallas_call(
        _stage1_body,
        grid=(N // BLK,),
        in_specs=[row, vec, vec,
                  full, vec, full, vec, full, vec,
                  full, vec, full, vec, full, vec],
        out_specs=[row] * 6,
        out_shape=[out] * 6,
    )(x, g1.reshape(1, D), bt1.reshape(1, D),
      Wq_l, bq_l.reshape(1, D), Wk_l, bk_l.reshape(1, D), Wv_l, bv_l.reshape(1, D),
      Wq_e, bq_e.reshape(1, D), Wk_e, bk_e.reshape(1, D), Wv_e, bv_e.reshape(1, D))


def _stage3_body(x_ref, wvl_ref, zl_ref, wve_ref, ze_ref, a_ref,
                 wol_ref, bol_ref, woe_ref, boe_ref,
                 w1_ref, b1_ref, w2_ref, b2_ref,
                 g2_ref, bt2_ref, g3_ref, bt3_ref, out_ref):
    f32 = jnp.float32
    sel = (jax.lax.broadcasted_iota(jnp.int32, (H, D), 1) // DH
           == jax.lax.broadcasted_iota(jnp.int32, (H, D), 0)).astype(f32)

    def norm_proj(wv_ref, z_ref, wo_ref, bo_ref):
        wv = wv_ref[0] + wv_ref[1]
        z = jnp.sum(z_ref[...], axis=0)
        zw = jnp.dot(z, sel, preferred_element_type=f32)
        o = wv / (zw + 1e-16)
        return jnp.dot(o, wo_ref[...], preferred_element_type=f32) + bo_ref[...]

    x_local = norm_proj(wvl_ref, zl_ref, wol_ref, bol_ref)
    x_exp = norm_proj(wve_ref, ze_ref, woe_ref, boe_ref)
    a = a_ref[0, 0]
    x = x_ref[...] + a * x_local + (1.0 - a) * x_exp

    mu = jnp.mean(x, axis=-1, keepdims=True)
    var = jnp.mean((x - mu) ** 2, axis=-1, keepdims=True)
    x = (x - mu) * jax.lax.rsqrt(var + 1e-5) * g2_ref[...] + bt2_ref[...]

    h = jax.nn.gelu(jnp.dot(x, w1_ref[...], preferred_element_type=f32) + b1_ref[...])
    x = x + jnp.dot(h, w2_ref[...], preferred_element_type=f32) + b2_ref[...]

    mu = jnp.mean(x, axis=-1, keepdims=True)
    var = jnp.mean((x - mu) ** 2, axis=-1, keepdims=True)
    out_ref[...] = (x - mu) * jax.lax.rsqrt(var + 1e-5) * g3_ref[...] + bt3_ref[...]


def _stage3(x, wv_l, z_l, wv_e, z_e, a_sig, Wo_l, bo_l, Wo_e, bo_e,
            W1, b1, W2, b2, g2, bt2, g3, bt3):
    row = pl.BlockSpec((BLK, D), lambda i: (i, 0))
    wvs = pl.BlockSpec((2, BLK, D), lambda i: (0, i, 0))
    zs = pl.BlockSpec((SC_CORES, BLK, H), lambda i: (0, i, 0))
    vec = pl.BlockSpec((1, D), lambda i: (0, 0))
    return pl.pallas_call(
        _stage3_body,
        grid=(N // BLK,),
        in_specs=[row, wvs, zs, wvs, zs,
                  pl.BlockSpec((1, 1), lambda i: (0, 0)),
                  pl.BlockSpec((D, D), lambda i: (0, 0)), vec,
                  pl.BlockSpec((D, D), lambda i: (0, 0)), vec,
                  pl.BlockSpec((D, 4 * D), lambda i: (0, 0)),
                  pl.BlockSpec((1, 4 * D), lambda i: (0, 0)),
                  pl.BlockSpec((4 * D, D), lambda i: (0, 0)), vec,
                  vec, vec, vec, vec],
        out_specs=row,
        out_shape=jax.ShapeDtypeStruct((N, D), jnp.float32),
    )(x, wv_l, z_l, wv_e, z_e, a_sig.reshape(1, 1),
      Wo_l, bo_l.reshape(1, D), Wo_e, bo_e.reshape(1, D),
      W1, b1.reshape(1, 4 * D), W2, b2.reshape(1, D),
      g2.reshape(1, D), bt2.reshape(1, D), g3.reshape(1, D), bt3.reshape(1, D))


def _edge_phase(q_hbm, k_hbm, v_hbm, dst_hbm, src_hbm, out_tbl,
                dsti, srci, zidxi, qrows, krows, vrows, zrows, table,
                sem_q, sem_k, sem_v, chunks_per_core, ci, sid):
    iters = (chunks_per_core + SC_SUBCORES - 1) // SC_SUBCORES
    zero16 = jnp.zeros((16,), jnp.float32)
    iota = lax.iota(jnp.int32, 16)

    @pl.loop(0, EC)
    def _(r):
        for j in range(D // 16):
            zrows[r, pl.ds(j * 16, 16)] = zero16

    @pl.loop(0, (TBL // EC + SC_SUBCORES - 1) // SC_SUBCORES)
    def _(i):
        strip = sid + i * SC_SUBCORES

        @pl.when(strip < TBL // EC)
        def _():
            pltpu.sync_copy(zrows, table.at[pl.ds(strip * EC, EC)])

    plsc.subcore_barrier()

    @pl.loop(0, iters)
    def _(it):
        c = sid + it * SC_SUBCORES

        @pl.when(c < chunks_per_core)
        def _():
            ebase = (ci * chunks_per_core + c) * EC
            pltpu.sync_copy(dst_hbm.at[pl.ds(ebase, EC)], dsti)
            pltpu.sync_copy(src_hbm.at[pl.ds(ebase, EC)], srci)
            cq = pltpu.async_copy(q_hbm.at[dsti], qrows, sem_q)
            ck = pltpu.async_copy(k_hbm.at[srci], krows, sem_k)
            cv = pltpu.async_copy(v_hbm.at[srci], vrows, sem_v)
            cq.wait()
            ck.wait()
            cv.wait()

            @pl.loop(0, EG)
            def _(g):
                rows = g * 16 + iota
                dvec = dsti[pl.ds(g * 16, 16)]
                zidxi[pl.ds(g * 16, 16)] = (
                    N + lax.shift_right_logical(dvec, 4))
                zcol0 = (dvec & 15) * 8
                dcols = [(iota + t) & 15 for t in range(DH)]
                for h in range(H):
                    b = h * DH
                    qs = [plsc.load_gather(qrows, [rows, dcols[t] + b])
                          for t in range(DH)]
                    ks = [plsc.load_gather(krows, [rows, dcols[t] + b])
                          for t in range(DH)]
                    ps = [q * k for q, k in zip(qs, ks)]
                    while len(ps) > 1:
                        ps = [a + c for a, c in zip(ps[::2], ps[1::2])]
                    w = jnp.exp(ps[0] * (1.0 / (DH ** 0.5)))
                    plsc.store_scatter(zrows, [rows, zcol0 + h], w)
                    vs = [plsc.load_gather(vrows, [rows, dcols[t] + b])
                          for t in range(DH)]
                    for t in range(DH):
                        plsc.store_scatter(vrows, [rows, dcols[t] + b],
                                           w * vs[t])

            pltpu.sync_copy(vrows, table.at[dsti], add=True)
            pltpu.sync_copy(zrows, table.at[zidxi], add=True)

            @pl.loop(0, EG)
            def _(g):
                rows = g * 16 + iota
                dvec = dsti[pl.ds(g * 16, 16)]
                zcol0 = (dvec & 15) * 8
                for h in range(H):
                    plsc.store_scatter(zrows, [rows, zcol0 + h], zero16)

    plsc.subcore_barrier()

    @pl.loop(0, (TBL // EC + SC_SUBCORES - 1) // SC_SUBCORES)
    def _(i):
        strip = sid + i * SC_SUBCORES

        @pl.when(strip < TBL // EC)
        def _():
            pltpu.sync_copy(table.at[pl.ds(strip * EC, EC)],
                            out_tbl.at[ci, pl.ds(strip * EC, EC)])


def _edges_body(ql, kl, vl, qe, ke, ve, dl, sl, de, se, otl, ote,
                dsti, srci, zidxi, qrows, krows, vrows, zrows, table,
                sem_q, sem_k, sem_v, chunks_l, chunks_e):
    ci = lax.axis_index("c")
    sid = lax.axis_index("s")
    scratch = (dsti, srci, zidxi, qrows, krows, vrows, zrows, table,
               sem_q, sem_k, sem_v)
    _edge_phase(ql, kl, vl, dl, sl, otl, *scratch, chunks_l, ci, sid)
    plsc.subcore_barrier()
    _edge_phase(qe, ke, ve, de, se, ote, *scratch, chunks_e, ci, sid)


def _edge_acc_sc(ql, kl, vl, qe, ke, ve, ei_l, ei_e, ne_l, ne_e):
    mesh = plsc.VectorSubcoreMesh(core_axis_name="c", subcore_axis_name="s")
    cp = pltpu.CompilerParams()
    if "needs_layout_passes" in pltpu.CompilerParams.__dataclass_fields__:
        cp = dataclasses.replace(cp, needs_layout_passes=False)

    body = functools.partial(_edges_body,
                             chunks_l=ne_l // SC_CORES // EC,
                             chunks_e=ne_e // SC_CORES // EC)
    tbl_t = jax.ShapeDtypeStruct((SC_CORES, TBL, D), jnp.float32)
    kern = pl.kernel(
        body,
        compiler_params=cp,
        out_type=[tbl_t, tbl_t],
        mesh=mesh,
        scratch_types=[
            pltpu.VMEM((EC,), jnp.int32),
            pltpu.VMEM((EC,), jnp.int32),
            pltpu.VMEM((EC,), jnp.int32),
            pltpu.VMEM((EC, D), jnp.float32),
            pltpu.VMEM((EC, D), jnp.float32),
            pltpu.VMEM((EC, D), jnp.float32),
            pltpu.VMEM((EC, D), jnp.float32),
            pltpu.VMEM_SHARED((TBL, D), jnp.float32),
            pltpu.SemaphoreType.DMA,
            pltpu.SemaphoreType.DMA,
            pltpu.SemaphoreType.DMA,
        ],
    )
    return kern(ql, kl, vl, qe, ke, ve, ei_l[1], ei_l[0], ei_e[1], ei_e[0])


def kernel(x, edge_index, expander_edge_index,
           Wq_l, bq_l, Wk_l, bk_l, Wv_l, bv_l, Wo_l, bo_l,
           Wq_e, bq_e, Wk_e, bk_e, Wv_e, bv_e, Wo_e, bo_e,
           W1, b1, W2, b2, g1, bt1, g2, bt2, g3, bt3, alpha):
    ql, kl, vl, qe, ke, ve = _stage1(
        x, g1, bt1, Wq_l, bq_l, Wk_l, bk_l, Wv_l, bv_l,
        Wq_e, bq_e, Wk_e, bk_e, Wv_e, bv_e)
    otl, ote = _edge_acc_sc(
        ql, kl, vl, qe, ke, ve, edge_index, expander_edge_index,
        320000, 40000)
    wv_l = otl[:, :N, :]
    wv_e = ote[:, :N, :]
    z_l = otl[:, N:N + ZPK, :].reshape(SC_CORES, N, H)
    z_e = ote[:, N:N + ZPK, :].reshape(SC_CORES, N, H)
    a_sig = jax.nn.sigmoid(alpha)
    return _stage3(x, wv_l, z_l, wv_e, z_e, a_sig, Wo_l, bo_l, Wo_e, bo_e,
                   W1, b1, W2, b2, g2, bt2, g3, bt3)
